# Initial kernel scaffold; baseline (speedup 1.0000x reference)
import functools

import jax
import jax.numpy as jnp
from jax import lax
from jax.experimental import pallas as pl
from jax.experimental.pallas import tpu as pltpu
from jax.experimental.pallas import tpu_sc as plsc

_LANES = 16
_CHUNK = 512
_IDX_MINOR = 128


def _comb_body(seg_ref, pe_ref, out_ref):
    out_ref[...] = seg_ref[:, None, :] + pe_ref[None, :, :]


def _make_comb(segment_table, pe_t):
    s, e = segment_table.shape
    t = pe_t.shape[0]
    return pl.pallas_call(
        _comb_body,
        out_shape=jax.ShapeDtypeStruct((s, t, e), jnp.float32),
    )(segment_table, pe_t)


@functools.lru_cache(maxsize=None)
def _sc_gather_fn(n_rows, t_len, embed):
    info = plsc.get_sparse_core_info()
    nw = info.num_cores * info.num_subcores
    nc = info.num_cores
    per_w = n_rows // nw
    n_chunks = per_w // _CHUNK
    r = _CHUNK // _IDX_MINOR

    @functools.partial(
        pl.kernel,
        mesh=plsc.VectorSubcoreMesh(core_axis_name="c", subcore_axis_name="s"),
        out_type=jax.ShapeDtypeStruct((n_rows, embed), jnp.float32),
        scratch_types=[
            pltpu.VMEM((r, _IDX_MINOR), jnp.int32),
            pltpu.VMEM((r, _IDX_MINOR), jnp.int32),
            pltpu.VMEM((r, _IDX_MINOR), jnp.int32),
            pltpu.VMEM((_CHUNK, embed), jnp.float32),
            pltpu.VMEM((_CHUNK, embed), jnp.float32),
            pltpu.SemaphoreType.DMA,
            pltpu.SemaphoreType.DMA,
        ],
    )
    def k(tok_hbm, comb_hbm, seq_hbm, seg_hbm, out_hbm,
          idx_v, seg_v, cidx_v, tok_b, comb_b, sem_t, sem_c):
        wid = lax.axis_index("s") * nc + lax.axis_index("c")
        wbase = wid * per_w

        def chunk(ci, carry):
            base = wbase + ci * _CHUNK
            r0 = base // _IDX_MINOR
            pltpu.sync_copy(seq_hbm.at[pl.ds(r0, r)], idx_v)
            pltpu.sync_copy(seg_hbm.at[pl.ds(r0, r)], seg_v)
            for rr in range(r):
                for kk in range(_IDX_MINOR // _LANES):
                    off = rr * _IDX_MINOR + kk * _LANES
                    sl = pl.ds(kk * _LANES, _LANES)
                    fvec = lax.broadcasted_iota(jnp.int32, (_LANES,), 0) + (base + off)
                    pvec = lax.rem(fvec, t_len)
                    cidx_v[rr, sl] = seg_v[rr, sl] * t_len + pvec
            cps = []
            for j in range(r):
                cps.append(pltpu.async_copy(
                    tok_hbm.at[idx_v.at[j]],
                    tok_b.at[pl.ds(j * _IDX_MINOR, _IDX_MINOR)], sem_t))
            for j in range(r):
                cps.append(pltpu.async_copy(
                    comb_hbm.at[cidx_v.at[j]],
                    comb_b.at[pl.ds(j * _IDX_MINOR, _IDX_MINOR)], sem_c))
            for cp in cps:
                cp.wait()

            def add_row(i, acc):
                for kk in range(embed // _LANES):
                    sl = pl.ds(kk * _LANES, _LANES)
                    tok_b[i, sl] = tok_b[i, sl] + comb_b[i, sl]
                return acc

            lax.fori_loop(0, _CHUNK, add_row, 0)
            pltpu.sync_copy(tok_b, out_hbm.at[pl.ds(base, _CHUNK)])
            return carry

        lax.fori_loop(0, n_chunks, chunk, 0)

    return k


def kernel(sequence, segment_label, token_table, segment_table, pe):
    b, t = sequence.shape
    embed = token_table.shape[1]
    n = b * t
    comb = _make_comb(segment_table, pe[:t])
    comb_flat = comb.reshape(segment_table.shape[0] * t, embed)
    seq2 = sequence.reshape(n // _IDX_MINOR, _IDX_MINOR)
    seg2 = segment_label.reshape(n // _IDX_MINOR, _IDX_MINOR)
    out = _sc_gather_fn(n, t, embed)(token_table, comb_flat, seq2, seg2)
    return out.reshape(b, t, embed)


# speedup vs baseline: 2.3134x; 2.3134x over previous
import functools

import jax
import jax.numpy as jnp
from jax import lax
from jax.experimental import pallas as pl
from jax.experimental.pallas import tpu as pltpu
from jax.experimental.pallas import tpu_sc as plsc

_LANES = 16
_CHUNK = 512
_IDX_MINOR = 128


def _comb_body(seg_ref, pe_ref, out_ref):
    seg = seg_ref[...]
    pe = pe_ref[...]
    out_ref[...] = seg[:, None, :] + pe[None, :, :]


def _make_comb(segment_table, pe_t):
    s, e = segment_table.shape
    t = pe_t.shape[0]
    return pl.pallas_call(
        _comb_body,
        out_shape=jax.ShapeDtypeStruct((s, t, e), jnp.float32),
    )(segment_table, pe_t)


@functools.lru_cache(maxsize=None)
def _sc_gather_fn(n_rows, t_len, embed):
    info = plsc.get_sparse_core_info()
    nw = info.num_cores * info.num_subcores
    nc = info.num_cores
    per_w = n_rows // nw
    n_super = per_w // (2 * _CHUNK)
    r = _CHUNK // _IDX_MINOR

    @functools.partial(
        pl.kernel,
        mesh=plsc.VectorSubcoreMesh(core_axis_name="c", subcore_axis_name="s"),
        compiler_params=pltpu.CompilerParams(use_tc_tiling_on_sc=False),
        out_type=jax.ShapeDtypeStruct((n_rows, embed), jnp.float32),
        scratch_types=[
            pltpu.VMEM((2 * r, _IDX_MINOR), jnp.int32),
            pltpu.VMEM((2 * r, _IDX_MINOR), jnp.int32),
            pltpu.VMEM((2 * r, _IDX_MINOR), jnp.int32),
            pltpu.VMEM((_CHUNK, embed), jnp.float32),
            pltpu.VMEM((_CHUNK, embed), jnp.float32),
            pltpu.SemaphoreType.DMA,
            pltpu.SemaphoreType.DMA,
        ],
    )
    def k(tok_hbm, comb_hbm, seq_hbm, seg_hbm, out_hbm,
          idx_v, seg_v, cidx_v, tok_b, comb_b, sem_t, sem_c):
        wid = lax.axis_index("s") * nc + lax.axis_index("c")
        wbase = wid * per_w

        def super_chunk(ci, carry):
            base_s = wbase + ci * 2 * _CHUNK
            r0 = pl.multiple_of(base_s // _IDX_MINOR, 8)
            pltpu.sync_copy(seq_hbm.at[pl.ds(r0, 2 * r)], idx_v)
            pltpu.sync_copy(seg_hbm.at[pl.ds(r0, 2 * r)], seg_v)
            for rr in range(2 * r):
                for kk in range(_IDX_MINOR // _LANES):
                    off = rr * _IDX_MINOR + kk * _LANES
                    sl = pl.ds(kk * _LANES, _LANES)
                    fvec = lax.broadcasted_iota(jnp.int32, (_LANES,), 0) + (base_s + off)
                    pvec = lax.rem(fvec, t_len)
                    cidx_v[rr, sl] = seg_v[rr, sl] * t_len + pvec
            for half in range(2):
                base = base_s + half * _CHUNK
                cps = []
                for j in range(r):
                    cps.append(pltpu.async_copy(
                        tok_hbm.at[idx_v.at[half * r + j]],
                        tok_b.at[pl.ds(j * _IDX_MINOR, _IDX_MINOR)], sem_t))
                for j in range(r):
                    cps.append(pltpu.async_copy(
                        comb_hbm.at[cidx_v.at[half * r + j]],
                        comb_b.at[pl.ds(j * _IDX_MINOR, _IDX_MINOR)], sem_c))
                for cp in cps:
                    cp.wait()

                def add_row(i, acc):
                    for kk in range(embed // _LANES):
                        sl = pl.ds(kk * _LANES, _LANES)
                        tok_b[i, sl] = tok_b[i, sl] + comb_b[i, sl]
                    return acc

                lax.fori_loop(0, _CHUNK, add_row, 0)
                pltpu.sync_copy(tok_b, out_hbm.at[pl.ds(base, _CHUNK)])
            return carry

        lax.fori_loop(0, n_super, super_chunk, 0)

    return k


def kernel(sequence, segment_label, token_table, segment_table, pe):
    b, t = sequence.shape
    embed = token_table.shape[1]
    n = b * t
    comb = _make_comb(segment_table, pe[:t])
    comb_flat = comb.reshape(segment_table.shape[0] * t, embed)
    seq2 = sequence.reshape(n // _IDX_MINOR, _IDX_MINOR)
    seg2 = segment_label.reshape(n // _IDX_MINOR, _IDX_MINOR)
    out = _sc_gather_fn(n, t, embed)(token_table, comb_flat, seq2, seg2)
    return out.reshape(b, t, embed)
